# baseline (speedup 1.0000x reference)
import jax
import jax.numpy as jnp
from jax.experimental import pallas as pl
from jax.experimental.pallas import tpu as pltpu

B, F, D, NC = 128, 768, 4096, 8192
BLOCK_NC = 1024
HALF = BLOCK_NC // 2
NBLK = NC // BLOCK_NC
NSTG = 4


def _body(x_ref, p_ref, c_ref, o_hbm, h_ref, stage, sems):
    k = pl.program_id(0)

    @pl.when(k == 0)
    def _encode():
        acc = jax.lax.dot_general(
            x_ref[...], p_ref[...], (((1,), (1,)), ((), ())),
            preferred_element_type=jnp.float32)
        h_ref[...] = jnp.sign(acc)

    def o_copy(step, n, slot):
        return pltpu.make_async_copy(
            stage.at[slot],
            o_hbm.at[:, pl.ds(step * BLOCK_NC + n * HALF, HALF)],
            sems.at[slot])

    for n in range(2):
        slot = jax.lax.rem(2 * k + n, NSTG)

        @pl.when(k >= 2)
        def _drain():
            o_copy(k - 2, n, slot).wait()

        stage[slot] = jax.lax.dot_general(
            h_ref[...], c_ref[pl.ds(n * HALF, HALF), :],
            (((1,), (1,)), ((), ())), preferred_element_type=jnp.float32)
        o_copy(k, n, slot).start()

    @pl.when(k == NBLK - 1)
    def _finish():
        for n in range(2):
            o_copy(k - 1, n, jax.lax.rem(2 * (k - 1) + n, NSTG)).wait()
        for n in range(2):
            o_copy(k, n, jax.lax.rem(2 * k + n, NSTG)).wait()


def kernel(x, projection, centroids):
    return pl.pallas_call(
        _body,
        grid=(NBLK,),
        in_specs=[
            pl.BlockSpec((B, F), lambda i: (0, 0)),
            pl.BlockSpec((D, F), lambda i: (0, 0)),
            pl.BlockSpec((BLOCK_NC, D), lambda i: (i, 0)),
        ],
        out_specs=pl.BlockSpec(memory_space=pltpu.MemorySpace.HBM),
        out_shape=jax.ShapeDtypeStruct((B, NC), jnp.float32),
        scratch_shapes=[
            pltpu.VMEM((B, D), jnp.float32),
            pltpu.VMEM((NSTG, B, HALF), jnp.float32),
            pltpu.SemaphoreType.DMA((NSTG,)),
        ],
    )(x, projection, centroids)


# speedup vs baseline: 1.0151x; 1.0151x over previous
import jax
import jax.numpy as jnp
from jax.experimental import pallas as pl
from jax.experimental.pallas import tpu as pltpu

B, F, D, NC = 128, 768, 4096, 8192
BLOCK_NC = 1024


def _body(x_ref, p_ref, c_ref, o_ref, h_ref):
    @pl.when(pl.program_id(0) == 0)
    def _encode():
        acc = jax.lax.dot_general(
            x_ref[...], p_ref[...], (((1,), (1,)), ((), ())),
            preferred_element_type=jnp.float32)
        h_ref[...] = jnp.sign(acc)

    dh = D // 2
    o_ref[...] = jax.lax.dot_general(
        h_ref[:, :dh], c_ref[:, :dh], (((1,), (1,)), ((), ())),
        preferred_element_type=jnp.float32) + jax.lax.dot_general(
        h_ref[:, dh:], c_ref[:, dh:], (((1,), (1,)), ((), ())),
        preferred_element_type=jnp.float32)


def kernel(x, projection, centroids):
    grid = (NC // BLOCK_NC,)
    return pl.pallas_call(
        _body,
        grid=grid,
        in_specs=[
            pl.BlockSpec((B, F), lambda i: (0, 0)),
            pl.BlockSpec((D, F), lambda i: (0, 0)),
            pl.BlockSpec((BLOCK_NC, D), lambda i: (i, 0)),
        ],
        out_specs=pl.BlockSpec((B, BLOCK_NC), lambda i: (0, i)),
        out_shape=jax.ShapeDtypeStruct((B, NC), jnp.float32),
        scratch_shapes=[pltpu.VMEM((B, D), jnp.float32)],
    )(x, projection, centroids)
